# baseline (speedup 1.0000x reference)
import functools

import jax
import jax.numpy as jnp
from jax.experimental import pallas as pl


def _celu(x):
    return jnp.where(x > 0, x, jnp.exp(jnp.minimum(x, 0.0)) - 1.0)


def _fused_kernel(sp_ref, x_ref, ws_ref, bs_ref, w1_ref, b1_ref, w2_ref,
                  b2_ref, out_ref, *, atoms_per_mol, mols_per_tile, nexp):
    tb = atoms_per_mol * mols_per_tile
    x = x_ref[...].reshape(tb, x_ref.shape[-1]).astype(jnp.bfloat16)
    shared = _celu(
        jnp.dot(x, ws_ref[...].astype(jnp.bfloat16),
                preferred_element_type=jnp.float32)
        + bs_ref[...])
    h = _celu(
        jnp.dot(shared.astype(jnp.bfloat16),
                w1_ref[...].astype(jnp.bfloat16),
                preferred_element_type=jnp.float32)
        + b1_ref[...])
    e_all = jnp.dot(h.astype(jnp.bfloat16), w2_ref[...],
                    preferred_element_type=jnp.float32) + b2_ref[...]
    sp = sp_ref[...]
    lane = jax.lax.broadcasted_iota(jnp.int32, (tb, nexp), 1)
    oh = (sp == lane).astype(jnp.float32)
    e = jnp.sum(e_all * oh, axis=1, keepdims=True)
    row = jax.lax.broadcasted_iota(jnp.int32, (tb, mols_per_tile), 0)
    col = jax.lax.broadcasted_iota(jnp.int32, (tb, mols_per_tile), 1)
    mask = (row // atoms_per_mol) == col
    out_ref[0, ...] = jnp.sum(jnp.where(mask, e, 0.0), axis=0,
                              keepdims=True)


def kernel(species, aev, W_shared, b_shared, W1, b1, W2, b2):
    bsz, natoms = species.shape
    n = bsz * natoms
    d = aev.shape[-1]
    nexp, ds, hdim = W1.shape

    mols_per_tile = 32
    tb = mols_per_tile * natoms
    grid = bsz // mols_per_tile

    sp_col = species.reshape(n, 1).astype(jnp.int32)
    w1c = jnp.transpose(W1, (1, 0, 2)).reshape(ds, nexp * hdim)
    b1c = b1.reshape(1, nexp * hdim)
    w2bd = ((W2[:, :, 0][:, :, None] *
             jnp.eye(nexp, dtype=W2.dtype)[:, None, :])
            .reshape(nexp * hdim, nexp).astype(jnp.bfloat16))
    b2v = b2.reshape(1, nexp)
    bsv = b_shared.reshape(1, ds)

    out = pl.pallas_call(
        functools.partial(_fused_kernel, atoms_per_mol=natoms,
                          mols_per_tile=mols_per_tile, nexp=nexp),
        grid=(grid,),
        in_specs=[
            pl.BlockSpec((tb, 1), lambda i: (i, 0)),
            pl.BlockSpec((mols_per_tile, natoms, d), lambda i: (i, 0, 0)),
            pl.BlockSpec((d, ds), lambda i: (0, 0)),
            pl.BlockSpec((1, ds), lambda i: (0, 0)),
            pl.BlockSpec((ds, nexp * hdim), lambda i: (0, 0)),
            pl.BlockSpec((1, nexp * hdim), lambda i: (0, 0)),
            pl.BlockSpec((nexp * hdim, nexp), lambda i: (0, 0)),
            pl.BlockSpec((1, nexp), lambda i: (0, 0)),
        ],
        out_specs=pl.BlockSpec((1, 1, mols_per_tile), lambda i: (i, 0, 0)),
        out_shape=jax.ShapeDtypeStruct((grid, 1, mols_per_tile), jnp.float32),
    )(sp_col, aev, W_shared, bsv, w1c, b1c, w2bd, b2v)

    energies = out.reshape(bsz)
    return (species, energies)


# speedup vs baseline: 1.0702x; 1.0702x over previous
import functools

import jax
import jax.numpy as jnp
from jax.experimental import pallas as pl


def _celu(x):
    return jnp.maximum(x, jnp.exp(jnp.minimum(x, 0.0)) - 1.0)


def _fused_kernel(oh_ref, x_ref, ws_ref, bs_ref, w1_ref, b1_ref, w2_ref,
                  b2_ref, out_ref, *, atoms_per_mol, mols_per_tile):
    x = x_ref[...].astype(jnp.bfloat16)
    shared = _celu(
        jnp.dot(x, ws_ref[...].astype(jnp.bfloat16),
                preferred_element_type=jnp.float32)
        + bs_ref[...])
    h = _celu(
        jnp.dot(shared.astype(jnp.bfloat16),
                w1_ref[...].astype(jnp.bfloat16),
                preferred_element_type=jnp.float32)
        + b1_ref[...])
    e_all = jnp.dot(h.astype(jnp.bfloat16), w2_ref[...],
                    preferred_element_type=jnp.float32) + b2_ref[...]
    e = jnp.sum(e_all * oh_ref[...], axis=1, keepdims=True)
    tb = e.shape[0]
    row = jax.lax.broadcasted_iota(jnp.int32, (tb, mols_per_tile), 0)
    col = jax.lax.broadcasted_iota(jnp.int32, (tb, mols_per_tile), 1)
    mask = (row // atoms_per_mol) == col
    out_ref[0, ...] = jnp.sum(jnp.where(mask, e, 0.0), axis=0,
                              keepdims=True)


def kernel(species, aev, W_shared, b_shared, W1, b1, W2, b2):
    bsz, natoms = species.shape
    n = bsz * natoms
    d = aev.shape[-1]
    nexp, ds, hdim = W1.shape

    tb = 2048
    mols_per_tile = tb // natoms
    grid = n // tb

    x = aev.reshape(n, d)
    onehot = (species.reshape(n, 1) ==
              jnp.arange(nexp, dtype=species.dtype)[None, :]).astype(jnp.float32)
    w1c = jnp.transpose(W1, (1, 0, 2)).reshape(ds, nexp * hdim)
    b1c = b1.reshape(1, nexp * hdim)
    w2bd = ((W2[:, :, 0][:, :, None] *
             jnp.eye(nexp, dtype=W2.dtype)[:, None, :])
            .reshape(nexp * hdim, nexp).astype(jnp.bfloat16))
    b2v = b2.reshape(1, nexp)
    bsv = b_shared.reshape(1, ds)

    out = pl.pallas_call(
        functools.partial(_fused_kernel, atoms_per_mol=natoms,
                          mols_per_tile=mols_per_tile),
        grid=(grid,),
        in_specs=[
            pl.BlockSpec((tb, nexp), lambda i: (i, 0)),
            pl.BlockSpec((tb, d), lambda i: (i, 0)),
            pl.BlockSpec((d, ds), lambda i: (0, 0)),
            pl.BlockSpec((1, ds), lambda i: (0, 0)),
            pl.BlockSpec((ds, nexp * hdim), lambda i: (0, 0)),
            pl.BlockSpec((1, nexp * hdim), lambda i: (0, 0)),
            pl.BlockSpec((nexp * hdim, nexp), lambda i: (0, 0)),
            pl.BlockSpec((1, nexp), lambda i: (0, 0)),
        ],
        out_specs=pl.BlockSpec((1, 1, mols_per_tile), lambda i: (i, 0, 0)),
        out_shape=jax.ShapeDtypeStruct((grid, 1, mols_per_tile), jnp.float32),
    )(onehot, x, W_shared, bsv, w1c, b1c, w2bd, b2v)

    energies = out.reshape(bsz)
    return (species, energies)


# speedup vs baseline: 1.1186x; 1.0452x over previous
import functools

import jax
import jax.numpy as jnp
from jax.experimental import pallas as pl


def _celu(x):
    return jnp.maximum(x, jnp.exp(jnp.minimum(x, 0.0)) - 1.0)


def _fused_kernel(oh_ref, x_ref, ws_ref, bs_ref, w1_ref, b1_ref, w2_ref,
                  b2_ref, out_ref, *, atoms_per_mol, mols_per_tile):
    x = x_ref[...].astype(jnp.bfloat16)
    shared = _celu(
        jnp.dot(x, ws_ref[...].astype(jnp.bfloat16),
                preferred_element_type=jnp.float32)
        + bs_ref[...])
    h = _celu(
        jnp.dot(shared.astype(jnp.bfloat16),
                w1_ref[...].astype(jnp.bfloat16),
                preferred_element_type=jnp.float32)
        + b1_ref[...])
    e_all = jnp.dot(h.astype(jnp.bfloat16), w2_ref[...],
                    preferred_element_type=jnp.float32) + b2_ref[...]
    e = jnp.sum(e_all * oh_ref[...], axis=1, keepdims=True)
    tb = e.shape[0]
    row = jax.lax.broadcasted_iota(jnp.int32, (tb, mols_per_tile), 0)
    col = jax.lax.broadcasted_iota(jnp.int32, (tb, mols_per_tile), 1)
    mask = (row // atoms_per_mol) == col
    out_ref[0, ...] = jnp.sum(jnp.where(mask, e, 0.0), axis=0,
                              keepdims=True)


def kernel(species, aev, W_shared, b_shared, W1, b1, W2, b2):
    bsz, natoms = species.shape
    n = bsz * natoms
    d = aev.shape[-1]
    nexp, ds, hdim = W1.shape

    tb = 4096
    mols_per_tile = tb // natoms
    grid = n // tb

    x = aev.reshape(n, d)
    onehot = (species.reshape(n, 1) ==
              jnp.arange(nexp, dtype=species.dtype)[None, :]).astype(jnp.float32)
    w1c = jnp.transpose(W1, (1, 0, 2)).reshape(ds, nexp * hdim)
    b1c = b1.reshape(1, nexp * hdim)
    w2bd = ((W2[:, :, 0][:, :, None] *
             jnp.eye(nexp, dtype=W2.dtype)[:, None, :])
            .reshape(nexp * hdim, nexp).astype(jnp.bfloat16))
    b2v = b2.reshape(1, nexp)
    bsv = b_shared.reshape(1, ds)

    out = pl.pallas_call(
        functools.partial(_fused_kernel, atoms_per_mol=natoms,
                          mols_per_tile=mols_per_tile),
        grid=(grid,),
        in_specs=[
            pl.BlockSpec((tb, nexp), lambda i: (i, 0)),
            pl.BlockSpec((tb, d), lambda i: (i, 0)),
            pl.BlockSpec((d, ds), lambda i: (0, 0)),
            pl.BlockSpec((1, ds), lambda i: (0, 0)),
            pl.BlockSpec((ds, nexp * hdim), lambda i: (0, 0)),
            pl.BlockSpec((1, nexp * hdim), lambda i: (0, 0)),
            pl.BlockSpec((nexp * hdim, nexp), lambda i: (0, 0)),
            pl.BlockSpec((1, nexp), lambda i: (0, 0)),
        ],
        out_specs=pl.BlockSpec((1, 1, mols_per_tile), lambda i: (i, 0, 0)),
        out_shape=jax.ShapeDtypeStruct((grid, 1, mols_per_tile), jnp.float32),
    )(onehot, x, W_shared, bsv, w1c, b1c, w2bd, b2v)

    energies = out.reshape(bsz)
    return (species, energies)


# speedup vs baseline: 1.1511x; 1.0291x over previous
import functools

import jax
import jax.numpy as jnp
from jax.experimental import pallas as pl


def _celu(x):
    return jnp.maximum(x, jnp.exp(jnp.minimum(x, 0.0)) - 1.0)


def _fused_kernel(oh_ref, x_ref, ws_ref, bs_ref, w1_ref, b1_ref, w2_ref,
                  b2_ref, out_ref, *, atoms_per_mol, mols_per_tile):
    x = x_ref[...].astype(jnp.bfloat16)
    shared = _celu(
        jnp.dot(x, ws_ref[...].astype(jnp.bfloat16),
                preferred_element_type=jnp.float32)
        + bs_ref[...])
    h = _celu(
        jnp.dot(shared.astype(jnp.bfloat16),
                w1_ref[...].astype(jnp.bfloat16),
                preferred_element_type=jnp.float32)
        + b1_ref[...])
    e_all = jnp.dot(h.astype(jnp.bfloat16), w2_ref[...],
                    preferred_element_type=jnp.float32) + b2_ref[...]
    e = jnp.sum(e_all * oh_ref[...], axis=1, keepdims=True)
    tb = e.shape[0]
    row = jax.lax.broadcasted_iota(jnp.int32, (tb, mols_per_tile), 0)
    col = jax.lax.broadcasted_iota(jnp.int32, (tb, mols_per_tile), 1)
    mask = (row // atoms_per_mol) == col
    out_ref[0, ...] = jnp.sum(jnp.where(mask, e, 0.0), axis=0,
                              keepdims=True)


def kernel(species, aev, W_shared, b_shared, W1, b1, W2, b2):
    bsz, natoms = species.shape
    n = bsz * natoms
    d = aev.shape[-1]
    nexp, ds, hdim = W1.shape

    tb = 8192
    mols_per_tile = tb // natoms
    grid = n // tb

    x = aev.reshape(n, d)
    onehot = (species.reshape(n, 1) ==
              jnp.arange(nexp, dtype=species.dtype)[None, :]).astype(jnp.float32)
    w1c = jnp.transpose(W1, (1, 0, 2)).reshape(ds, nexp * hdim)
    b1c = b1.reshape(1, nexp * hdim)
    w2bd = ((W2[:, :, 0][:, :, None] *
             jnp.eye(nexp, dtype=W2.dtype)[:, None, :])
            .reshape(nexp * hdim, nexp).astype(jnp.bfloat16))
    b2v = b2.reshape(1, nexp)
    bsv = b_shared.reshape(1, ds)

    out = pl.pallas_call(
        functools.partial(_fused_kernel, atoms_per_mol=natoms,
                          mols_per_tile=mols_per_tile),
        grid=(grid,),
        in_specs=[
            pl.BlockSpec((tb, nexp), lambda i: (i, 0)),
            pl.BlockSpec((tb, d), lambda i: (i, 0)),
            pl.BlockSpec((d, ds), lambda i: (0, 0)),
            pl.BlockSpec((1, ds), lambda i: (0, 0)),
            pl.BlockSpec((ds, nexp * hdim), lambda i: (0, 0)),
            pl.BlockSpec((1, nexp * hdim), lambda i: (0, 0)),
            pl.BlockSpec((nexp * hdim, nexp), lambda i: (0, 0)),
            pl.BlockSpec((1, nexp), lambda i: (0, 0)),
        ],
        out_specs=pl.BlockSpec((1, 1, mols_per_tile), lambda i: (i, 0, 0)),
        out_shape=jax.ShapeDtypeStruct((grid, 1, mols_per_tile), jnp.float32),
    )(onehot, x, W_shared, bsv, w1c, b1c, w2bd, b2v)

    energies = out.reshape(bsz)
    return (species, energies)


# speedup vs baseline: 1.2080x; 1.0495x over previous
import functools

import jax
import jax.numpy as jnp
from jax.experimental import pallas as pl


def _celu(x):
    return jnp.maximum(x, jnp.exp(jnp.minimum(x, 0.0)) - 1.0)


def _fused_kernel(oh_ref, x_ref, ws_ref, bs_ref, w1_ref, b1_ref, w2_ref,
                  b2_ref, out_ref, *, atoms_per_mol, mols_per_tile):
    x = x_ref[...].astype(jnp.bfloat16)
    shared = _celu(
        jnp.dot(x, ws_ref[...].astype(jnp.bfloat16),
                preferred_element_type=jnp.float32)
        + bs_ref[...])
    h = _celu(
        (jnp.dot(shared.astype(jnp.bfloat16),
                 w1_ref[...].astype(jnp.bfloat16),
                 preferred_element_type=jnp.float32)
         + b1_ref[...]).astype(jnp.bfloat16))
    e_all = jnp.dot(h, w2_ref[...],
                    preferred_element_type=jnp.float32) + b2_ref[...]
    e = jnp.sum(e_all * oh_ref[...], axis=1, keepdims=True)
    tb = e.shape[0]
    row = jax.lax.broadcasted_iota(jnp.int32, (tb, mols_per_tile), 0)
    col = jax.lax.broadcasted_iota(jnp.int32, (tb, mols_per_tile), 1)
    mask = (row // atoms_per_mol) == col
    out_ref[0, ...] = jnp.sum(jnp.where(mask, e, 0.0), axis=0,
                              keepdims=True)


def kernel(species, aev, W_shared, b_shared, W1, b1, W2, b2):
    bsz, natoms = species.shape
    n = bsz * natoms
    d = aev.shape[-1]
    nexp, ds, hdim = W1.shape

    tb = 8192
    mols_per_tile = tb // natoms
    grid = n // tb

    x = aev.reshape(n, d)
    onehot = (species.reshape(n, 1) ==
              jnp.arange(nexp, dtype=species.dtype)[None, :]).astype(jnp.float32)
    w1c = jnp.transpose(W1, (1, 0, 2)).reshape(ds, nexp * hdim)
    b1c = b1.reshape(1, nexp * hdim)
    w2bd = ((W2[:, :, 0][:, :, None] *
             jnp.eye(nexp, dtype=W2.dtype)[:, None, :])
            .reshape(nexp * hdim, nexp).astype(jnp.bfloat16))
    b2v = b2.reshape(1, nexp)
    bsv = b_shared.reshape(1, ds)

    out = pl.pallas_call(
        functools.partial(_fused_kernel, atoms_per_mol=natoms,
                          mols_per_tile=mols_per_tile),
        grid=(grid,),
        in_specs=[
            pl.BlockSpec((tb, nexp), lambda i: (i, 0)),
            pl.BlockSpec((tb, d), lambda i: (i, 0)),
            pl.BlockSpec((d, ds), lambda i: (0, 0)),
            pl.BlockSpec((1, ds), lambda i: (0, 0)),
            pl.BlockSpec((ds, nexp * hdim), lambda i: (0, 0)),
            pl.BlockSpec((1, nexp * hdim), lambda i: (0, 0)),
            pl.BlockSpec((nexp * hdim, nexp), lambda i: (0, 0)),
            pl.BlockSpec((1, nexp), lambda i: (0, 0)),
        ],
        out_specs=pl.BlockSpec((1, 1, mols_per_tile), lambda i: (i, 0, 0)),
        out_shape=jax.ShapeDtypeStruct((grid, 1, mols_per_tile), jnp.float32),
    )(onehot, x, W_shared, bsv, w1c, b1c, w2bd, b2v)

    energies = out.reshape(bsz)
    return (species, energies)


# speedup vs baseline: 1.2247x; 1.0138x over previous
import functools

import jax
import jax.numpy as jnp
from jax.experimental import pallas as pl


def _celu(x):
    return jnp.maximum(x, jnp.exp(jnp.minimum(x, 0.0)) - 1.0)


def _fused_kernel(oh_ref, x_ref, ws_ref, bs_ref, w1_ref, b1_ref, w2_ref,
                  b2_ref, out_ref, *, atoms_per_mol, mols_per_tile):
    x = x_ref[...].astype(jnp.bfloat16)
    shared = _celu(
        (jnp.dot(x, ws_ref[...].astype(jnp.bfloat16),
                 preferred_element_type=jnp.float32)
         + bs_ref[...]).astype(jnp.bfloat16))
    h = _celu(
        (jnp.dot(shared, w1_ref[...].astype(jnp.bfloat16),
                 preferred_element_type=jnp.float32)
         + b1_ref[...]).astype(jnp.bfloat16))
    e_all = jnp.dot(h, w2_ref[...],
                    preferred_element_type=jnp.float32) + b2_ref[...]
    e = jnp.sum(e_all * oh_ref[...], axis=1, keepdims=True)
    tb = e.shape[0]
    row = jax.lax.broadcasted_iota(jnp.int32, (tb, mols_per_tile), 0)
    col = jax.lax.broadcasted_iota(jnp.int32, (tb, mols_per_tile), 1)
    mask = (row // atoms_per_mol) == col
    out_ref[0, ...] = jnp.sum(jnp.where(mask, e, 0.0), axis=0,
                              keepdims=True)


def kernel(species, aev, W_shared, b_shared, W1, b1, W2, b2):
    bsz, natoms = species.shape
    n = bsz * natoms
    d = aev.shape[-1]
    nexp, ds, hdim = W1.shape

    tb = 8192
    mols_per_tile = tb // natoms
    grid = n // tb

    x = aev.reshape(n, d)
    onehot = (species.reshape(n, 1) ==
              jnp.arange(nexp, dtype=species.dtype)[None, :]).astype(jnp.float32)
    w1c = jnp.transpose(W1, (1, 0, 2)).reshape(ds, nexp * hdim)
    b1c = b1.reshape(1, nexp * hdim)
    w2bd = ((W2[:, :, 0][:, :, None] *
             jnp.eye(nexp, dtype=W2.dtype)[:, None, :])
            .reshape(nexp * hdim, nexp).astype(jnp.bfloat16))
    b2v = b2.reshape(1, nexp)
    bsv = b_shared.reshape(1, ds)

    out = pl.pallas_call(
        functools.partial(_fused_kernel, atoms_per_mol=natoms,
                          mols_per_tile=mols_per_tile),
        grid=(grid,),
        in_specs=[
            pl.BlockSpec((tb, nexp), lambda i: (i, 0)),
            pl.BlockSpec((tb, d), lambda i: (i, 0)),
            pl.BlockSpec((d, ds), lambda i: (0, 0)),
            pl.BlockSpec((1, ds), lambda i: (0, 0)),
            pl.BlockSpec((ds, nexp * hdim), lambda i: (0, 0)),
            pl.BlockSpec((1, nexp * hdim), lambda i: (0, 0)),
            pl.BlockSpec((nexp * hdim, nexp), lambda i: (0, 0)),
            pl.BlockSpec((1, nexp), lambda i: (0, 0)),
        ],
        out_specs=pl.BlockSpec((1, 1, mols_per_tile), lambda i: (i, 0, 0)),
        out_shape=jax.ShapeDtypeStruct((grid, 1, mols_per_tile), jnp.float32),
    )(onehot, x, W_shared, bsv, w1c, b1c, w2bd, b2v)

    energies = out.reshape(bsz)
    return (species, energies)
